# baseline (speedup 1.0000x reference)
import functools

import jax
import jax.numpy as jnp
from jax import lax
from jax.experimental import pallas as pl
from jax.experimental.pallas import tpu as pltpu
from jax.experimental.pallas import tpu_sc as plsc

B = 2
T = 2048
S = 2048
H = 12
K = 204
NSUB = 16
SLOTS = 14
PAIRS = NSUB * SLOTS
LANES = 16


def _prep_body(te_ref, fore_ref, tgt_ref, tok_ref, sw_ref):
    t_iota_c = lax.broadcasted_iota(jnp.int32, (T, 1), 0)
    t_iota_r = lax.broadcasted_iota(jnp.int32, (1, T), 1)
    for b in range(B):
        te = te_ref[b]
        fv = fore_ref[b].reshape(1, -1)
        num = lax.dot_general(te, fv, (((1,), (1,)), ((), ())),
                              preferred_element_type=jnp.float32)
        tnorm = jnp.sqrt(jnp.sum(te * te, axis=1, keepdims=True))
        fnorm = jnp.sqrt(jnp.sum(fv * fv))
        w = num / jnp.maximum(tnorm * fnorm, 1e-8)
        tgt = tgt_ref[b].reshape(T, 1)
        mask = (tgt > 0) | (t_iota_c == 0)
        wm = jnp.where(mask, w, -1.0)
        seq_len = jnp.sum(mask.astype(jnp.float32))
        kk = jnp.minimum(jnp.ceil(seq_len * 0.1), float(K))

        eye = (t_iota_c == t_iota_r).astype(jnp.float32)
        wm_row = lax.dot_general(wm, eye, (((0,), (0,)), ((), ())),
                                 preferred_element_type=jnp.float32)

        gt = wm > wm_row
        eq = wm == wm_row
        ult = t_iota_c < t_iota_r
        rank = jnp.sum((gt | (eq & ult)).astype(jnp.float32), axis=0,
                       keepdims=True)

        p_iota = lax.broadcasted_iota(
            jnp.int32, (NSUB * SLOTS, 1), 0).astype(jnp.float32)
        R = ((rank == p_iota) & (p_iota < kk)).astype(jnp.float32)
        tw = jnp.concatenate(
            [t_iota_c.astype(jnp.float32),
             wm * (1.0 / (float(H) * kk))], axis=1)
        res = lax.dot_general(R, tw, (((1,), (0,)), ((), ())),
                              preferred_element_type=jnp.float32)
        tok = res[:, 0:1].astype(jnp.int32)
        h_iota = lax.broadcasted_iota(jnp.int32, (1, H), 1)
        tok_ref[b] = tok + T * h_iota + (H * T) * b
        sw_ref[b] = jnp.broadcast_to(res[:, 1:2], (NSUB * SLOTS, LANES))


def _prep(target_embed, fore, tgt):
    return pl.pallas_call(
        _prep_body,
        out_shape=[
            jax.ShapeDtypeStruct((B, NSUB * SLOTS, H), jnp.int32),
            jax.ShapeDtypeStruct((B, NSUB * SLOTS, LANES), jnp.float32),
        ],
    )(target_embed, fore, tgt)


def _sc_body(a_hbm, rows_hbm, sw_hbm, z_hbm, out_hbm,
             idx_v, sw_v, buf0, buf1, buf2, acc, shared, zidx,
             sem0, sem1, sem2):
    c = lax.axis_index("c")
    s = lax.axis_index("s")
    pltpu.sync_copy(z_hbm, zidx)
    pltpu.sync_copy(rows_hbm.at[c, s], idx_v)
    pltpu.sync_copy(sw_hbm.at[c, s], sw_v)
    bufs = (buf0, buf1, buf2)
    sems = (sem0, sem1, sem2)
    nbuf = len(bufs)
    cps = {}
    for j in range(nbuf - 1):
        cps[j] = pltpu.async_copy(a_hbm.at[idx_v.at[j]], bufs[j % nbuf],
                                  sems[j % nbuf])
    for j in range(SLOTS):
        if j + nbuf - 1 < SLOTS:
            jj = j + nbuf - 1
            cps[jj] = pltpu.async_copy(a_hbm.at[idx_v.at[jj]],
                                       bufs[jj % nbuf], sems[jj % nbuf])
        cps.pop(j).wait()
        buf = bufs[j % nbuf]
        swv = sw_v[j]

        @plsc.parallel_loop(0, S // LANES, 1, unroll=8)
        def _chunk(i, buf=buf, swv=swv, first=(j == 0)):
            sl = pl.ds(i * LANES, LANES)
            tot = buf[0, sl]
            for h in range(1, H):
                tot = tot + buf[h, sl]
            v = jnp.maximum(tot * swv, 0.0)
            if first:
                acc[0, sl] = v
            else:
                acc[0, sl] = acc[0, sl] + v
    @pl.when(s == 0)
    def _seed():
        pltpu.sync_copy(acc, shared)

    plsc.subcore_barrier()

    @pl.when(s != 0)
    def _add():
        pltpu.sync_copy(acc, shared.at[zidx], add=True)

    plsc.subcore_barrier()

    @pl.when(s == 0)
    def _normalize():
        pltpu.sync_copy(shared, acc)
        nchunk = S // LANES
        lane = lax.iota(jnp.int32, LANES)

        def _bcast_min(v):
            for k in (1, 2, 4, 8):
                v = jnp.minimum(v, v.at[lane ^ k].get(mode="promise_in_bounds"))
            return v

        def _bcast_max(v):
            for k in (1, 2, 4, 8):
                v = jnp.maximum(v, v.at[lane ^ k].get(mode="promise_in_bounds"))
            return v

        def _minb(i, m):
            return jnp.minimum(m, acc[0, pl.ds(i * LANES, LANES)])

        mn = _bcast_min(lax.fori_loop(1, nchunk, _minb, acc[0, pl.ds(0, LANES)]))

        def _maxb(i, m):
            sl = pl.ds(i * LANES, LANES)
            cmi = acc[0, sl] - mn
            buf0[0, sl] = cmi
            return jnp.maximum(m, cmi)

        cm0 = acc[0, pl.ds(0, LANES)] - mn
        buf0[0, pl.ds(0, LANES)] = cm0
        mx = _bcast_max(lax.fori_loop(1, nchunk, _maxb, cm0))
        mx = jnp.where(mx < 1e-12, 1e-12, mx)

        @plsc.parallel_loop(0, nchunk, 1, unroll=4)
        def _div(i):
            sl = pl.ds(i * LANES, LANES)
            buf0[1, sl] = buf0[0, sl] / mx

        pltpu.sync_copy(buf0.at[1], out_hbm.at[c])


_sc_gather = functools.partial(
    pl.kernel,
    mesh=plsc.VectorSubcoreMesh(core_axis_name="c", subcore_axis_name="s"),
    out_type=jax.ShapeDtypeStruct((B, S), jnp.float32),
    scratch_types=[
        pltpu.VMEM((SLOTS, H), jnp.int32),
        pltpu.VMEM((SLOTS, LANES), jnp.float32),
        pltpu.VMEM((H, S), jnp.float32),
        pltpu.VMEM((H, S), jnp.float32),
        pltpu.VMEM((H, S), jnp.float32),
        pltpu.VMEM((1, S), jnp.float32),
        pltpu.VMEM_SHARED((1, S), jnp.float32),
        pltpu.VMEM((1,), jnp.int32),
        pltpu.SemaphoreType.DMA,
        pltpu.SemaphoreType.DMA,
        pltpu.SemaphoreType.DMA,
    ],
)(_sc_body)


def kernel(fore_rep_encoded, target_embed, align_attns, targets):
    tgt = targets[:, :-1]
    rows, sw16 = _prep(target_embed, fore_rep_encoded, tgt)
    rows_hbm = rows.reshape(B, NSUB, SLOTS, H)
    sw_hbm = sw16.reshape(B, NSUB, SLOTS, LANES)
    a2d = align_attns[0].reshape(B * H * T, S)
    zeros1 = jnp.zeros((1,), jnp.int32)
    return _sc_gather(a2d, rows_hbm, sw_hbm, zeros1)


# speedup vs baseline: 1.0472x; 1.0472x over previous
import functools

import jax
import jax.numpy as jnp
from jax import lax
from jax.experimental import pallas as pl
from jax.experimental.pallas import tpu as pltpu
from jax.experimental.pallas import tpu_sc as plsc

B = 2
T = 2048
S = 2048
H = 12
K = 204
NSUB = 16
SLOTS = 13
PAIRS = NSUB * SLOTS
LANES = 16


def _prep_body(te_ref, fore_ref, tgt_ref, tok_ref, sw_ref):
    t_iota_c = lax.broadcasted_iota(jnp.int32, (T, 1), 0)
    t_iota_r = lax.broadcasted_iota(jnp.int32, (1, T), 1)
    for b in range(B):
        te = te_ref[b]
        fv = fore_ref[b].reshape(1, -1)
        num = lax.dot_general(te, fv, (((1,), (1,)), ((), ())),
                              preferred_element_type=jnp.float32)
        tnorm = jnp.sqrt(jnp.sum(te * te, axis=1, keepdims=True))
        fnorm = jnp.sqrt(jnp.sum(fv * fv))
        w = num / jnp.maximum(tnorm * fnorm, 1e-8)
        tgt = tgt_ref[b].reshape(T, 1)
        mask = (tgt > 0) | (t_iota_c == 0)
        wm = jnp.where(mask, w, -1.0)
        seq_len = jnp.sum(mask.astype(jnp.float32))
        kk = jnp.minimum(jnp.ceil(seq_len * 0.1), float(K))

        eye = (t_iota_c == t_iota_r).astype(jnp.float32)
        wm_row = lax.dot_general(wm, eye, (((0,), (0,)), ((), ())),
                                 preferred_element_type=jnp.float32)

        gt = wm > wm_row
        eq = wm == wm_row
        ult = t_iota_c < t_iota_r
        rank = jnp.sum((gt | (eq & ult)).astype(jnp.float32), axis=0,
                       keepdims=True)

        p_iota = lax.broadcasted_iota(
            jnp.int32, (NSUB * SLOTS, 1), 0).astype(jnp.float32)
        R = ((rank == p_iota) & (p_iota < kk)).astype(jnp.float32)
        tw = jnp.concatenate(
            [t_iota_c.astype(jnp.float32),
             wm * (1.0 / (float(H) * kk))], axis=1)
        res = lax.dot_general(R, tw, (((1,), (0,)), ((), ())),
                              preferred_element_type=jnp.float32)
        tok = res[:, 0:1].astype(jnp.int32)
        h_iota = lax.broadcasted_iota(jnp.int32, (1, H), 1)
        tok_ref[b] = tok + T * h_iota + (H * T) * b
        sw_ref[b] = jnp.broadcast_to(res[:, 1:2], (NSUB * SLOTS, LANES))


def _prep(target_embed, fore, tgt):
    return pl.pallas_call(
        _prep_body,
        out_shape=[
            jax.ShapeDtypeStruct((B, NSUB * SLOTS, H), jnp.int32),
            jax.ShapeDtypeStruct((B, NSUB * SLOTS, LANES), jnp.float32),
        ],
    )(target_embed, fore, tgt)


def _sc_body(a_hbm, rows_hbm, sw_hbm, z_hbm, out_hbm,
             idx_v, sw_v, buf0, buf1, buf2, acc, shared, zidx,
             sem0, sem1, sem2):
    c = lax.axis_index("c")
    s = lax.axis_index("s")
    pltpu.sync_copy(z_hbm, zidx)
    pltpu.sync_copy(rows_hbm.at[c, s], idx_v)
    pltpu.sync_copy(sw_hbm.at[c, s], sw_v)
    bufs = (buf0, buf1, buf2)
    sems = (sem0, sem1, sem2)
    nbuf = len(bufs)
    cps = {}
    for j in range(nbuf - 1):
        cps[j] = pltpu.async_copy(a_hbm.at[idx_v.at[j]], bufs[j % nbuf],
                                  sems[j % nbuf])
    for j in range(SLOTS):
        if j + nbuf - 1 < SLOTS:
            jj = j + nbuf - 1
            cps[jj] = pltpu.async_copy(a_hbm.at[idx_v.at[jj]],
                                       bufs[jj % nbuf], sems[jj % nbuf])
        cps.pop(j).wait()
        buf = bufs[j % nbuf]
        swv = sw_v[j]

        @plsc.parallel_loop(0, S // LANES, 1, unroll=4)
        def _chunk(i, buf=buf, swv=swv, first=(j == 0)):
            sl = pl.ds(i * LANES, LANES)
            tot = buf[0, sl]
            for h in range(1, H):
                tot = tot + buf[h, sl]
            v = jnp.maximum(tot * swv, 0.0)
            if first:
                acc[0, sl] = v
            else:
                acc[0, sl] = acc[0, sl] + v
    @pl.when(s == 0)
    def _seed():
        pltpu.sync_copy(acc, shared)

    plsc.subcore_barrier()

    @pl.when(s != 0)
    def _add():
        pltpu.sync_copy(acc, shared.at[zidx], add=True)

    plsc.subcore_barrier()

    @pl.when(s == 0)
    def _normalize():
        pltpu.sync_copy(shared, acc)
        nchunk = S // LANES
        lane = lax.iota(jnp.int32, LANES)

        def _bcast_min(v):
            for k in (1, 2, 4, 8):
                v = jnp.minimum(v, v.at[lane ^ k].get(mode="promise_in_bounds"))
            return v

        def _bcast_max(v):
            for k in (1, 2, 4, 8):
                v = jnp.maximum(v, v.at[lane ^ k].get(mode="promise_in_bounds"))
            return v

        def _minb(i, m):
            return jnp.minimum(m, acc[0, pl.ds(i * LANES, LANES)])

        mn = _bcast_min(lax.fori_loop(1, nchunk, _minb, acc[0, pl.ds(0, LANES)]))

        def _maxb(i, m):
            sl = pl.ds(i * LANES, LANES)
            cmi = acc[0, sl] - mn
            buf0[0, sl] = cmi
            return jnp.maximum(m, cmi)

        cm0 = acc[0, pl.ds(0, LANES)] - mn
        buf0[0, pl.ds(0, LANES)] = cm0
        mx = _bcast_max(lax.fori_loop(1, nchunk, _maxb, cm0))
        mx = jnp.where(mx < 1e-12, 1e-12, mx)

        @plsc.parallel_loop(0, nchunk, 1, unroll=4)
        def _div(i):
            sl = pl.ds(i * LANES, LANES)
            buf0[1, sl] = buf0[0, sl] / mx

        pltpu.sync_copy(buf0.at[1], out_hbm.at[c])


_sc_gather = functools.partial(
    pl.kernel,
    mesh=plsc.VectorSubcoreMesh(core_axis_name="c", subcore_axis_name="s"),
    out_type=jax.ShapeDtypeStruct((B, S), jnp.float32),
    scratch_types=[
        pltpu.VMEM((SLOTS, H), jnp.int32),
        pltpu.VMEM((SLOTS, LANES), jnp.float32),
        pltpu.VMEM((H, S), jnp.float32),
        pltpu.VMEM((H, S), jnp.float32),
        pltpu.VMEM((H, S), jnp.float32),
        pltpu.VMEM((1, S), jnp.float32),
        pltpu.VMEM_SHARED((1, S), jnp.float32),
        pltpu.VMEM((1,), jnp.int32),
        pltpu.SemaphoreType.DMA,
        pltpu.SemaphoreType.DMA,
        pltpu.SemaphoreType.DMA,
    ],
)(_sc_body)


def kernel(fore_rep_encoded, target_embed, align_attns, targets):
    tgt = targets[:, :-1]
    rows, sw16 = _prep(target_embed, fore_rep_encoded, tgt)
    rows_hbm = rows.reshape(B, NSUB, SLOTS, H)
    sw_hbm = sw16.reshape(B, NSUB, SLOTS, LANES)
    a2d = align_attns[0].reshape(B * H * T, S)
    zeros1 = jnp.zeros((1,), jnp.int32)
    return _sc_gather(a2d, rows_hbm, sw_hbm, zeros1)


# speedup vs baseline: 1.1907x; 1.1369x over previous
import functools

import jax
import jax.numpy as jnp
from jax import lax
from jax.experimental import pallas as pl
from jax.experimental.pallas import tpu as pltpu
from jax.experimental.pallas import tpu_sc as plsc

B = 2
T = 2048
S = 2048
H = 12
K = 204
NSUB = 16
SLOTS = 13
PAIRS = NSUB * SLOTS
LANES = 16


def _prep_body(te_ref, fore_ref, tgt_ref, tok_ref, sw_ref):
    t_iota_c = lax.broadcasted_iota(jnp.int32, (T, 1), 0)
    t_iota_r = lax.broadcasted_iota(jnp.int32, (1, T), 1)
    for b in range(B):
        te = te_ref[b]
        fv = fore_ref[b].reshape(1, -1)
        num = lax.dot_general(te, fv, (((1,), (1,)), ((), ())),
                              preferred_element_type=jnp.float32)
        tnorm = jnp.sqrt(jnp.sum(te * te, axis=1, keepdims=True))
        fnorm = jnp.sqrt(jnp.sum(fv * fv))
        w = num / jnp.maximum(tnorm * fnorm, 1e-8)
        tgt = tgt_ref[b].reshape(T, 1)
        mask = (tgt > 0) | (t_iota_c == 0)
        wm = jnp.where(mask, w, -1.0)
        seq_len = jnp.sum(mask.astype(jnp.float32))
        kk = jnp.minimum(jnp.ceil(seq_len * 0.1), float(K))

        wm_row = wm.reshape(1, T)

        gt = wm > wm_row
        eq = wm == wm_row
        ult = t_iota_c < t_iota_r
        rank = jnp.sum((gt | (eq & ult)).astype(jnp.float32), axis=0,
                       keepdims=True)

        p_iota = lax.broadcasted_iota(
            jnp.int32, (NSUB * SLOTS, 1), 0).astype(jnp.float32)
        R = ((rank == p_iota) & (p_iota < kk)).astype(jnp.float32)
        tw = jnp.concatenate(
            [t_iota_c.astype(jnp.float32),
             wm * (1.0 / (float(H) * kk))], axis=1)
        res = lax.dot_general(R, tw, (((1,), (0,)), ((), ())),
                              preferred_element_type=jnp.float32)
        tok = res[:, 0:1].astype(jnp.int32)
        h_iota = lax.broadcasted_iota(jnp.int32, (1, H), 1)
        tok_ref[b] = tok + T * h_iota + (H * T) * b
        sw_ref[b] = jnp.broadcast_to(res[:, 1:2], (NSUB * SLOTS, LANES))


def _prep(target_embed, fore, tgt):
    return pl.pallas_call(
        _prep_body,
        out_shape=[
            jax.ShapeDtypeStruct((B, NSUB * SLOTS, H), jnp.int32),
            jax.ShapeDtypeStruct((B, NSUB * SLOTS, LANES), jnp.float32),
        ],
    )(target_embed, fore, tgt)


def _sc_body(a_hbm, rows_hbm, sw_hbm, z_hbm, out_hbm,
             idx_v, sw_v, buf0, buf1, buf2, acc, shared, zidx,
             sem0, sem1, sem2):
    c = lax.axis_index("c")
    s = lax.axis_index("s")
    pltpu.sync_copy(z_hbm, zidx)
    pltpu.sync_copy(rows_hbm.at[c, s], idx_v)
    pltpu.sync_copy(sw_hbm.at[c, s], sw_v)
    bufs = (buf0, buf1, buf2)
    sems = (sem0, sem1, sem2)
    nbuf = len(bufs)
    cps = {}
    for j in range(nbuf - 1):
        cps[j] = pltpu.async_copy(a_hbm.at[idx_v.at[j]], bufs[j % nbuf],
                                  sems[j % nbuf])
    for j in range(SLOTS):
        if j + nbuf - 1 < SLOTS:
            jj = j + nbuf - 1
            cps[jj] = pltpu.async_copy(a_hbm.at[idx_v.at[jj]],
                                       bufs[jj % nbuf], sems[jj % nbuf])
        cps.pop(j).wait()
        buf = bufs[j % nbuf]
        swv = sw_v[j]

        @plsc.parallel_loop(0, S // LANES, 1, unroll=4)
        def _chunk(i, buf=buf, swv=swv, first=(j == 0)):
            sl = pl.ds(i * LANES, LANES)
            tot = buf[0, sl]
            for h in range(1, H):
                tot = tot + buf[h, sl]
            v = jnp.maximum(tot * swv, 0.0)
            if first:
                acc[0, sl] = v
            else:
                acc[0, sl] = acc[0, sl] + v
    @pl.when(s == 0)
    def _seed():
        pltpu.sync_copy(acc, shared)

    plsc.subcore_barrier()

    @pl.when(s != 0)
    def _add():
        pltpu.sync_copy(acc, shared.at[zidx], add=True)

    plsc.subcore_barrier()

    @pl.when(s == 0)
    def _normalize():
        pltpu.sync_copy(shared, acc)
        nchunk = S // LANES
        lane = lax.iota(jnp.int32, LANES)

        def _bcast_min(v):
            for k in (1, 2, 4, 8):
                v = jnp.minimum(v, v.at[lane ^ k].get(mode="promise_in_bounds"))
            return v

        def _bcast_max(v):
            for k in (1, 2, 4, 8):
                v = jnp.maximum(v, v.at[lane ^ k].get(mode="promise_in_bounds"))
            return v

        def _minb(i, m):
            return jnp.minimum(m, acc[0, pl.ds(i * LANES, LANES)])

        mn = _bcast_min(lax.fori_loop(1, nchunk, _minb, acc[0, pl.ds(0, LANES)]))

        def _maxb(i, m):
            sl = pl.ds(i * LANES, LANES)
            cmi = acc[0, sl] - mn
            buf0[0, sl] = cmi
            return jnp.maximum(m, cmi)

        cm0 = acc[0, pl.ds(0, LANES)] - mn
        buf0[0, pl.ds(0, LANES)] = cm0
        mx = _bcast_max(lax.fori_loop(1, nchunk, _maxb, cm0))
        mx = jnp.where(mx < 1e-12, 1e-12, mx)

        @plsc.parallel_loop(0, nchunk, 1, unroll=4)
        def _div(i):
            sl = pl.ds(i * LANES, LANES)
            buf0[1, sl] = buf0[0, sl] / mx

        pltpu.sync_copy(buf0.at[1], out_hbm.at[c])


_sc_gather = functools.partial(
    pl.kernel,
    mesh=plsc.VectorSubcoreMesh(core_axis_name="c", subcore_axis_name="s"),
    out_type=jax.ShapeDtypeStruct((B, S), jnp.float32),
    scratch_types=[
        pltpu.VMEM((SLOTS, H), jnp.int32),
        pltpu.VMEM((SLOTS, LANES), jnp.float32),
        pltpu.VMEM((H, S), jnp.float32),
        pltpu.VMEM((H, S), jnp.float32),
        pltpu.VMEM((H, S), jnp.float32),
        pltpu.VMEM((1, S), jnp.float32),
        pltpu.VMEM_SHARED((1, S), jnp.float32),
        pltpu.VMEM((1,), jnp.int32),
        pltpu.SemaphoreType.DMA,
        pltpu.SemaphoreType.DMA,
        pltpu.SemaphoreType.DMA,
    ],
)(_sc_body)


def kernel(fore_rep_encoded, target_embed, align_attns, targets):
    tgt = targets[:, :-1]
    rows, sw16 = _prep(target_embed, fore_rep_encoded, tgt)
    rows_hbm = rows.reshape(B, NSUB, SLOTS, H)
    sw_hbm = sw16.reshape(B, NSUB, SLOTS, LANES)
    a2d = align_attns[0].reshape(B * H * T, S)
    zeros1 = jnp.zeros((1,), jnp.int32)
    return _sc_gather(a2d, rows_hbm, sw_hbm, zeros1)
